# baseline (speedup 1.0000x reference)
import functools

import jax
import jax.numpy as jnp
from jax import lax
from jax.experimental import pallas as pl
from jax.experimental.pallas import tpu as pltpu
from jax.experimental.pallas import tpu_sc as plsc

_BLOCK_ROWS = 256
_NC, _NS, _L = 2, 16, 16
_NW = _NC * _NS
_CHUNK = 128


def _copy_body(x_ref, o_ref):
    o_ref[...] = x_ref[...]


@functools.lru_cache(maxsize=None)
def _make_sc_patch(n):
    per_w = n // _NW
    chunk = _CHUNK if per_w % _CHUNK == 0 else per_w
    n_chunks = per_w // chunk

    @functools.partial(
        pl.kernel,
        out_type=(),
        mesh=plsc.VectorSubcoreMesh(core_axis_name="c", subcore_axis_name="s"),
        scratch_types=[
            pltpu.VMEM((n_chunks, chunk, chunk), jnp.float32),
            pltpu.VMEM((_L,), jnp.float32),
            pltpu.SemaphoreType.DMA,
        ],
    )
    def sc_patch(y_hbm, sig_hbm, blocks_v, sig_v, sem):
        wid = lax.axis_index("s") * _NC + lax.axis_index("c")
        pltpu.sync_copy(sig_hbm, sig_v)
        sig = sig_v[...]
        lane = lax.iota(jnp.int32, _L)
        gathers = []
        for c in range(n_chunks):
            base = wid * per_w + c * chunk
            gathers.append(
                pltpu.async_copy(
                    y_hbm.at[pl.ds(base, chunk), pl.ds(base, chunk)],
                    blocks_v.at[c],
                    sem,
                )
            )
        for cp in gathers:
            cp.wait()
        scatters = []
        for c in range(n_chunks):
            for k in range(chunk):
                s0 = (k // _L) * _L
                vec = blocks_v[c, k, pl.ds(s0, _L)]
                blocks_v[c, k, pl.ds(s0, _L)] = vec + jnp.where(
                    lane == k - s0, sig, jnp.float32(0.0)
                )
            base = wid * per_w + c * chunk
            scatters.append(
                pltpu.async_copy(
                    blocks_v.at[c],
                    y_hbm.at[pl.ds(base, chunk), pl.ds(base, chunk)],
                    sem,
                )
            )
        for cp in scatters:
            cp.wait()

    return sc_patch


def kernel(x, sigma2):
    n, m = x.shape
    br = _BLOCK_ROWS if n % _BLOCK_ROWS == 0 else n
    y = pl.pallas_call(
        _copy_body,
        grid=(n // br,),
        in_specs=[pl.BlockSpec((br, m), lambda i: (i, 0))],
        out_specs=pl.BlockSpec((br, m), lambda i: (i, 0)),
        out_shape=jax.ShapeDtypeStruct((n, m), x.dtype),
        compiler_params=pltpu.CompilerParams(
            dimension_semantics=("parallel",),
        ),
    )(x)
    sig16 = jnp.broadcast_to(sigma2.astype(x.dtype), (_L,))
    y_ref = jax.new_ref(y)
    _make_sc_patch(n)(y_ref, sig16)
    return y_ref[...]
